# Initial kernel scaffold; baseline (speedup 1.0000x reference)
import jax
import jax.numpy as jnp
from jax.experimental import pallas as pl


def kernel(inputs, position_embeddings):
    raise NotImplementedError("write your pallas kernel here")


# speedup vs baseline: 5.0527x; 5.0527x over previous
import jax
import jax.numpy as jnp
from jax.experimental import pallas as pl


def _bcast_body(tab_ref, out_ref):
    out_ref[...] = jnp.broadcast_to(tab_ref[...][None, :, :], out_ref.shape)


def kernel(inputs, position_embeddings):
    B, S, H = inputs.shape
    table = position_embeddings[:S]
    BLK = 512
    return pl.pallas_call(
        _bcast_body,
        grid=(S // BLK,),
        in_specs=[pl.BlockSpec((BLK, H), lambda i: (i, 0))],
        out_specs=pl.BlockSpec((B, BLK, H), lambda i: (0, i, 0)),
        out_shape=jax.ShapeDtypeStruct((B, S, H), jnp.float32),
    )(table)
